# baseline (speedup 1.0000x reference)
import functools

import jax
import jax.numpy as jnp
from jax.experimental import pallas as pl


_TN = 1024
_BN_EPS = 1e-5


def _bf_split3(x):
    h = x.astype(jnp.bfloat16)
    r = x - h.astype(jnp.float32)
    l = r.astype(jnp.bfloat16)
    r2 = r - l.astype(jnp.float32)
    return h, l, r2.astype(jnp.bfloat16)


def _pass1_kernel(unk_ref, known_ref, kf_ref, uf_ref, w1_ref,
                  x1_ref, acc_ref, *, m):
    b = pl.program_id(0)
    t = pl.program_id(1)

    tn = unk_ref.shape[1]
    kraw = known_ref[0]
    uraw = jnp.transpose(unk_ref[0], (1, 0))

    kb = (kraw * -2.0).astype(jnp.bfloat16)
    ub = uraw.astype(jnp.bfloat16)
    k2 = jnp.sum(kraw * kraw, axis=1, keepdims=True)
    u2 = jnp.sum(uraw * uraw, axis=0, keepdims=True)
    k2h, k2l, k2l2 = _bf_split3(k2)
    u2h, u2l, u2l2 = _bf_split3(u2)
    ones_m = jnp.ones((m, 3), dtype=jnp.bfloat16)
    ones_t = jnp.ones((3, tn), dtype=jnp.bfloat16)
    amat = jnp.concatenate([kb, k2h, k2l, k2l2, ones_m], axis=1)
    bmat = jnp.concatenate(
        [ub, ones_t,
         jnp.concatenate([u2h, u2l, u2l2], axis=0)], axis=0)
    d2 = jnp.maximum(
        jnp.dot(amat, bmat, preferred_element_type=jnp.float32), 0.0)

    lowmask = jnp.int32(0x000003FF)
    bias = jnp.int32(0x00800000)
    miota = jax.lax.broadcasted_iota(jnp.int32, (m, tn), 0) + bias
    dbits = jax.lax.bitcast_convert_type(d2, jnp.int32)
    key = jax.lax.bitcast_convert_type((dbits & ~lowmask) + miota,
                                       jnp.float32)

    big = jnp.float32(jnp.inf)
    h = m // 2
    p0 = jnp.minimum(key[:h], key[h:])
    p1 = jnp.maximum(key[:h], key[h:])
    q = h // 2
    a0, b0 = p0[:q], p0[q:]
    a1, b1 = p1[:q], p1[q:]
    t0 = jnp.minimum(a0, b0)
    mm1 = jnp.maximum(a0, b0)
    nn1 = jnp.minimum(a1, b1)
    t1 = jnp.minimum(mm1, nn1)
    t2 = jnp.maximum(mm1, nn1)
    while t0.shape[0] > 8:
        q = t0.shape[0] // 2
        a0, b0 = t0[:q], t0[q:]
        a1, b1 = t1[:q], t1[q:]
        a2, b2 = t2[:q], t2[q:]
        c0 = jnp.minimum(a0, b0)
        mm1 = jnp.maximum(a0, b0)
        nn1 = jnp.minimum(a1, b1)
        c1 = jnp.minimum(mm1, nn1)
        mm2 = jnp.maximum(mm1, nn1)
        nn2 = jnp.minimum(a2, b2)
        c2 = jnp.minimum(mm2, nn2)
        t0, t1, t2 = c0, c1, c2
    allc = jnp.concatenate([t0, t1, t2], axis=0)
    k0 = jnp.min(allc, axis=0, keepdims=True)
    allc1 = jnp.where(allc == k0, big, allc)
    k1 = jnp.min(allc1, axis=0, keepdims=True)
    allc2 = jnp.where(allc1 == k1, big, allc1)
    k2s = jnp.min(allc2, axis=0, keepdims=True)

    def _unkey(k):
        kb_ = jax.lax.bitcast_convert_type(k, jnp.int32) - bias
        return jax.lax.bitcast_convert_type(kb_ & ~lowmask, jnp.float32)

    r0 = 1.0 / (_unkey(k0) + 1e-8)
    r1 = 1.0 / (_unkey(k1) + 1e-8)
    r2 = 1.0 / (_unkey(k2s) + 1e-8)
    rnorm = 1.0 / (r0 + r1 + r2)

    st = jnp.where(key == k0, r0 * rnorm, jnp.float32(0.0))
    st = jnp.where(key == k1, r1 * rnorm, st)
    st = jnp.where(key == k2s, r2 * rnorm, st)

    interp = jnp.dot(kf_ref[0].astype(jnp.bfloat16), st.astype(jnp.bfloat16),
                     preferred_element_type=jnp.float32)
    f = jnp.concatenate([interp.astype(jnp.bfloat16),
                         uf_ref[0].astype(jnp.bfloat16)], axis=0)
    w1b = w1_ref[...].astype(jnp.bfloat16)
    x1 = jnp.dot(w1b, f, preferred_element_type=jnp.float32)
    x1b = x1.astype(jnp.bfloat16)
    x1_ref[0] = x1b

    @pl.when(jnp.logical_and(b == 0, t == 0))
    def _init():
        acc_ref[...] = jnp.zeros_like(acc_ref)

    ones = jnp.ones((tn, 1), dtype=jnp.bfloat16)
    rs = jnp.dot(x1b, ones, preferred_element_type=jnp.float32)
    rss = jnp.dot(x1b * x1b, ones, preferred_element_type=jnp.float32)
    acc_ref[...] += jnp.concatenate([rs, rss], axis=1)


def _bn_coeffs_inkernel(acc, g, bb, ntot):
    mean = acc[:, 0:1] * (1.0 / ntot)
    var = acc[:, 1:2] * (1.0 / ntot) - mean * mean
    sc = g * jax.lax.rsqrt(var + _BN_EPS)
    bi = bb - mean * sc
    return sc, bi


def _pass2_kernel(x1_ref, acc1_ref, g_ref, b_ref, w2_ref,
                  x2_ref, acc_ref, *, ntot):
    b = pl.program_id(0)
    t = pl.program_id(1)
    sc, bi = _bn_coeffs_inkernel(acc1_ref[...], g_ref[...], b_ref[...], ntot)
    y = jnp.maximum(x1_ref[0].astype(jnp.float32) * sc + bi, 0.0)
    w2b = w2_ref[...].astype(jnp.bfloat16)
    x2 = jnp.dot(w2b, y.astype(jnp.bfloat16),
                 preferred_element_type=jnp.float32)
    x2b = x2.astype(jnp.bfloat16)
    x2_ref[0] = x2b

    @pl.when(jnp.logical_and(b == 0, t == 0))
    def _init():
        acc_ref[...] = jnp.zeros_like(acc_ref)

    ones = jnp.ones((x2.shape[1], 1), dtype=jnp.bfloat16)
    rs = jnp.dot(x2b, ones, preferred_element_type=jnp.float32)
    rss = jnp.dot(x2b * x2b, ones, preferred_element_type=jnp.float32)
    acc_ref[...] += jnp.concatenate([rs, rss], axis=1)


def _pass3_kernel(x2_ref, acc2_ref, g_ref, b_ref, out_ref, *, ntot):
    sc, bi = _bn_coeffs_inkernel(acc2_ref[...], g_ref[...], b_ref[...], ntot)
    out_ref[0] = jnp.maximum(x2_ref[0].astype(jnp.float32) * sc + bi, 0.0)


def kernel(unknown, known, unknow_feats, known_feats, W1, g1, b1, W2, g2, b2):
    B, n, _ = unknown.shape
    m = known.shape[1]
    c2 = known_feats.shape[1]
    c1 = unknow_feats.shape[1]
    co1 = W1.shape[0]
    co2 = W2.shape[0]
    tn = _TN
    nt = n // tn
    ntot = float(B * n)

    g1r = g1.reshape(co1, 1)
    b1r = b1.reshape(co1, 1)
    g2r = g2.reshape(co2, 1)
    b2r = b2.reshape(co2, 1)

    grid = (B, nt)
    x1, acc1 = pl.pallas_call(
        functools.partial(_pass1_kernel, m=m),
        grid=grid,
        in_specs=[
            pl.BlockSpec((1, tn, 3), lambda b, t: (b, t, 0)),
            pl.BlockSpec((1, m, 3), lambda b, t: (b, 0, 0)),
            pl.BlockSpec((1, c2, m), lambda b, t: (b, 0, 0)),
            pl.BlockSpec((1, c1, tn), lambda b, t: (b, 0, t)),
            pl.BlockSpec((co1, c1 + c2), lambda b, t: (0, 0)),
        ],
        out_specs=[
            pl.BlockSpec((1, co1, tn), lambda b, t: (b, 0, t)),
            pl.BlockSpec((co1, 2), lambda b, t: (0, 0)),
        ],
        out_shape=[
            jax.ShapeDtypeStruct((B, co1, n), jnp.bfloat16),
            jax.ShapeDtypeStruct((co1, 2), jnp.float32),
        ],
    )(unknown, known, known_feats, unknow_feats, W1)

    x2, acc2 = pl.pallas_call(
        functools.partial(_pass2_kernel, ntot=ntot),
        grid=grid,
        in_specs=[
            pl.BlockSpec((1, co1, tn), lambda b, t: (b, 0, t)),
            pl.BlockSpec((co1, 2), lambda b, t: (0, 0)),
            pl.BlockSpec((co1, 1), lambda b, t: (0, 0)),
            pl.BlockSpec((co1, 1), lambda b, t: (0, 0)),
            pl.BlockSpec((co2, co1), lambda b, t: (0, 0)),
        ],
        out_specs=[
            pl.BlockSpec((1, co2, tn), lambda b, t: (b, 0, t)),
            pl.BlockSpec((co2, 2), lambda b, t: (0, 0)),
        ],
        out_shape=[
            jax.ShapeDtypeStruct((B, co2, n), jnp.bfloat16),
            jax.ShapeDtypeStruct((co2, 2), jnp.float32),
        ],
    )(x1, acc1, g1r, b1r, W2)

    out = pl.pallas_call(
        functools.partial(_pass3_kernel, ntot=ntot),
        grid=grid,
        in_specs=[
            pl.BlockSpec((1, co2, tn), lambda b, t: (b, 0, t)),
            pl.BlockSpec((co2, 2), lambda b, t: (0, 0)),
            pl.BlockSpec((co2, 1), lambda b, t: (0, 0)),
            pl.BlockSpec((co2, 1), lambda b, t: (0, 0)),
        ],
        out_specs=pl.BlockSpec((1, co2, tn), lambda b, t: (b, 0, t)),
        out_shape=jax.ShapeDtypeStruct((B, co2, n), jnp.float32),
    )(x2, acc2, g2r, b2r)

    return out


# speedup vs baseline: 1.1550x; 1.1550x over previous
import functools

import jax
import jax.numpy as jnp
from jax.experimental import pallas as pl


_TN = 2048
_BN_EPS = 1e-5


def _bf_split3(x):
    h = x.astype(jnp.bfloat16)
    r = x - h.astype(jnp.float32)
    l = r.astype(jnp.bfloat16)
    r2 = r - l.astype(jnp.float32)
    return h, l, r2.astype(jnp.bfloat16)


def _pass1_kernel(unk_ref, known_ref, kf_ref, uf_ref, w1_ref,
                  x1_ref, acc_ref, *, m):
    b = pl.program_id(0)
    t = pl.program_id(1)

    tn = unk_ref.shape[1]
    kraw = known_ref[0]
    uraw = jnp.transpose(unk_ref[0], (1, 0))

    kb = (kraw * -2.0).astype(jnp.bfloat16)
    ub = uraw.astype(jnp.bfloat16)
    k2 = jnp.sum(kraw * kraw, axis=1, keepdims=True)
    u2 = jnp.sum(uraw * uraw, axis=0, keepdims=True)
    k2h, k2l, k2l2 = _bf_split3(k2)
    u2h, u2l, u2l2 = _bf_split3(u2)
    ones_m = jnp.ones((m, 3), dtype=jnp.bfloat16)
    ones_t = jnp.ones((3, tn), dtype=jnp.bfloat16)
    amat = jnp.concatenate([kb, k2h, k2l, k2l2, ones_m], axis=1)
    bmat = jnp.concatenate(
        [ub, ones_t,
         jnp.concatenate([u2h, u2l, u2l2], axis=0)], axis=0)
    d2 = jnp.maximum(
        jnp.dot(amat, bmat, preferred_element_type=jnp.float32), 0.0)

    lowmask = jnp.int32(0x000003FF)
    bias = jnp.int32(0x00800000)
    miota = jax.lax.broadcasted_iota(jnp.int32, (m, tn), 0) + bias
    dbits = jax.lax.bitcast_convert_type(d2, jnp.int32)
    key = jax.lax.bitcast_convert_type((dbits & ~lowmask) + miota,
                                       jnp.float32)

    big = jnp.float32(jnp.inf)
    h = m // 2
    p0 = jnp.minimum(key[:h], key[h:])
    p1 = jnp.maximum(key[:h], key[h:])
    q = h // 2
    a0, b0 = p0[:q], p0[q:]
    a1, b1 = p1[:q], p1[q:]
    t0 = jnp.minimum(a0, b0)
    mm1 = jnp.maximum(a0, b0)
    nn1 = jnp.minimum(a1, b1)
    t1 = jnp.minimum(mm1, nn1)
    t2 = jnp.maximum(mm1, nn1)
    while t0.shape[0] > 8:
        q = t0.shape[0] // 2
        a0, b0 = t0[:q], t0[q:]
        a1, b1 = t1[:q], t1[q:]
        a2, b2 = t2[:q], t2[q:]
        c0 = jnp.minimum(a0, b0)
        mm1 = jnp.maximum(a0, b0)
        nn1 = jnp.minimum(a1, b1)
        c1 = jnp.minimum(mm1, nn1)
        mm2 = jnp.maximum(mm1, nn1)
        nn2 = jnp.minimum(a2, b2)
        c2 = jnp.minimum(mm2, nn2)
        t0, t1, t2 = c0, c1, c2
    allc = jnp.concatenate([t0, t1, t2], axis=0)
    k0 = jnp.min(allc, axis=0, keepdims=True)
    allc1 = jnp.where(allc == k0, big, allc)
    k1 = jnp.min(allc1, axis=0, keepdims=True)
    allc2 = jnp.where(allc1 == k1, big, allc1)
    k2s = jnp.min(allc2, axis=0, keepdims=True)

    def _unkey(k):
        kb_ = jax.lax.bitcast_convert_type(k, jnp.int32) - bias
        return jax.lax.bitcast_convert_type(kb_ & ~lowmask, jnp.float32)

    r0 = 1.0 / (_unkey(k0) + 1e-8)
    r1 = 1.0 / (_unkey(k1) + 1e-8)
    r2 = 1.0 / (_unkey(k2s) + 1e-8)
    rnorm = 1.0 / (r0 + r1 + r2)

    st = jnp.where(key == k0, r0 * rnorm, jnp.float32(0.0))
    st = jnp.where(key == k1, r1 * rnorm, st)
    st = jnp.where(key == k2s, r2 * rnorm, st)

    interp = jnp.dot(kf_ref[0].astype(jnp.bfloat16), st.astype(jnp.bfloat16),
                     preferred_element_type=jnp.float32)
    f = jnp.concatenate([interp.astype(jnp.bfloat16),
                         uf_ref[0].astype(jnp.bfloat16)], axis=0)
    w1b = w1_ref[...].astype(jnp.bfloat16)
    x1 = jnp.dot(w1b, f, preferred_element_type=jnp.float32)
    x1b = x1.astype(jnp.bfloat16)
    x1_ref[0] = x1b

    @pl.when(jnp.logical_and(b == 0, t == 0))
    def _init():
        acc_ref[...] = jnp.zeros_like(acc_ref)

    ones = jnp.ones((tn, 1), dtype=jnp.bfloat16)
    rs = jnp.dot(x1b, ones, preferred_element_type=jnp.float32)
    rss = jnp.dot(x1b * x1b, ones, preferred_element_type=jnp.float32)
    acc_ref[...] += jnp.concatenate([rs, rss], axis=1)


def _bn_coeffs_inkernel(acc, g, bb, ntot):
    mean = acc[:, 0:1] * (1.0 / ntot)
    var = acc[:, 1:2] * (1.0 / ntot) - mean * mean
    sc = g * jax.lax.rsqrt(var + _BN_EPS)
    bi = bb - mean * sc
    return sc, bi


def _pass2_kernel(x1_ref, acc1_ref, g_ref, b_ref, w2_ref,
                  x2_ref, acc_ref, *, ntot):
    b = pl.program_id(0)
    t = pl.program_id(1)
    sc, bi = _bn_coeffs_inkernel(acc1_ref[...], g_ref[...], b_ref[...], ntot)
    y = jnp.maximum(x1_ref[0].astype(jnp.float32) * sc + bi, 0.0)
    w2b = w2_ref[...].astype(jnp.bfloat16)
    x2 = jnp.dot(w2b, y.astype(jnp.bfloat16),
                 preferred_element_type=jnp.float32)
    x2b = x2.astype(jnp.bfloat16)
    x2_ref[0] = x2b

    @pl.when(jnp.logical_and(b == 0, t == 0))
    def _init():
        acc_ref[...] = jnp.zeros_like(acc_ref)

    ones = jnp.ones((x2.shape[1], 1), dtype=jnp.bfloat16)
    rs = jnp.dot(x2b, ones, preferred_element_type=jnp.float32)
    rss = jnp.dot(x2b * x2b, ones, preferred_element_type=jnp.float32)
    acc_ref[...] += jnp.concatenate([rs, rss], axis=1)


def _pass3_kernel(x2_ref, acc2_ref, g_ref, b_ref, out_ref, *, ntot):
    sc, bi = _bn_coeffs_inkernel(acc2_ref[...], g_ref[...], b_ref[...], ntot)
    out_ref[0] = jnp.maximum(x2_ref[0].astype(jnp.float32) * sc + bi, 0.0)


def kernel(unknown, known, unknow_feats, known_feats, W1, g1, b1, W2, g2, b2):
    B, n, _ = unknown.shape
    m = known.shape[1]
    c2 = known_feats.shape[1]
    c1 = unknow_feats.shape[1]
    co1 = W1.shape[0]
    co2 = W2.shape[0]
    tn = _TN
    nt = n // tn
    ntot = float(B * n)

    g1r = g1.reshape(co1, 1)
    b1r = b1.reshape(co1, 1)
    g2r = g2.reshape(co2, 1)
    b2r = b2.reshape(co2, 1)

    grid = (B, nt)
    x1, acc1 = pl.pallas_call(
        functools.partial(_pass1_kernel, m=m),
        grid=grid,
        in_specs=[
            pl.BlockSpec((1, tn, 3), lambda b, t: (b, t, 0)),
            pl.BlockSpec((1, m, 3), lambda b, t: (b, 0, 0)),
            pl.BlockSpec((1, c2, m), lambda b, t: (b, 0, 0)),
            pl.BlockSpec((1, c1, tn), lambda b, t: (b, 0, t)),
            pl.BlockSpec((co1, c1 + c2), lambda b, t: (0, 0)),
        ],
        out_specs=[
            pl.BlockSpec((1, co1, tn), lambda b, t: (b, 0, t)),
            pl.BlockSpec((co1, 2), lambda b, t: (0, 0)),
        ],
        out_shape=[
            jax.ShapeDtypeStruct((B, co1, n), jnp.bfloat16),
            jax.ShapeDtypeStruct((co1, 2), jnp.float32),
        ],
    )(unknown, known, known_feats, unknow_feats, W1)

    x2, acc2 = pl.pallas_call(
        functools.partial(_pass2_kernel, ntot=ntot),
        grid=grid,
        in_specs=[
            pl.BlockSpec((1, co1, tn), lambda b, t: (b, 0, t)),
            pl.BlockSpec((co1, 2), lambda b, t: (0, 0)),
            pl.BlockSpec((co1, 1), lambda b, t: (0, 0)),
            pl.BlockSpec((co1, 1), lambda b, t: (0, 0)),
            pl.BlockSpec((co2, co1), lambda b, t: (0, 0)),
        ],
        out_specs=[
            pl.BlockSpec((1, co2, tn), lambda b, t: (b, 0, t)),
            pl.BlockSpec((co2, 2), lambda b, t: (0, 0)),
        ],
        out_shape=[
            jax.ShapeDtypeStruct((B, co2, n), jnp.bfloat16),
            jax.ShapeDtypeStruct((co2, 2), jnp.float32),
        ],
    )(x1, acc1, g1r, b1r, W2)

    out = pl.pallas_call(
        functools.partial(_pass3_kernel, ntot=ntot),
        grid=grid,
        in_specs=[
            pl.BlockSpec((1, co2, tn), lambda b, t: (b, 0, t)),
            pl.BlockSpec((co2, 2), lambda b, t: (0, 0)),
            pl.BlockSpec((co2, 1), lambda b, t: (0, 0)),
            pl.BlockSpec((co2, 1), lambda b, t: (0, 0)),
        ],
        out_specs=pl.BlockSpec((1, co2, tn), lambda b, t: (b, 0, t)),
        out_shape=jax.ShapeDtypeStruct((B, co2, n), jnp.float32),
    )(x2, acc2, g2r, b2r)

    return out


# speedup vs baseline: 1.2491x; 1.0815x over previous
import functools

import jax
import jax.numpy as jnp
from jax.experimental import pallas as pl


_TN = 4096
_BN_EPS = 1e-5


def _bf_split3(x):
    h = x.astype(jnp.bfloat16)
    r = x - h.astype(jnp.float32)
    l = r.astype(jnp.bfloat16)
    r2 = r - l.astype(jnp.float32)
    return h, l, r2.astype(jnp.bfloat16)


def _pass1_kernel(unk_ref, known_ref, kf_ref, uf_ref, w1_ref,
                  x1_ref, acc_ref, *, m):
    b = pl.program_id(0)
    t = pl.program_id(1)

    tn = unk_ref.shape[1]
    kraw = known_ref[0]
    uraw = jnp.transpose(unk_ref[0], (1, 0))

    kb = (kraw * -2.0).astype(jnp.bfloat16)
    ub = uraw.astype(jnp.bfloat16)
    k2 = jnp.sum(kraw * kraw, axis=1, keepdims=True)
    u2 = jnp.sum(uraw * uraw, axis=0, keepdims=True)
    k2h, k2l, k2l2 = _bf_split3(k2)
    u2h, u2l, u2l2 = _bf_split3(u2)
    ones_m = jnp.ones((m, 3), dtype=jnp.bfloat16)
    ones_t = jnp.ones((3, tn), dtype=jnp.bfloat16)
    amat = jnp.concatenate([kb, k2h, k2l, k2l2, ones_m], axis=1)
    bmat = jnp.concatenate(
        [ub, ones_t,
         jnp.concatenate([u2h, u2l, u2l2], axis=0)], axis=0)
    d2 = jnp.maximum(
        jnp.dot(amat, bmat, preferred_element_type=jnp.float32), 0.0)

    lowmask = jnp.int32(0x000003FF)
    bias = jnp.int32(0x00800000)
    miota = jax.lax.broadcasted_iota(jnp.int32, (m, tn), 0) + bias
    dbits = jax.lax.bitcast_convert_type(d2, jnp.int32)
    key = jax.lax.bitcast_convert_type((dbits & ~lowmask) + miota,
                                       jnp.float32)

    big = jnp.float32(jnp.inf)
    h = m // 2
    p0 = jnp.minimum(key[:h], key[h:])
    p1 = jnp.maximum(key[:h], key[h:])
    q = h // 2
    a0, b0 = p0[:q], p0[q:]
    a1, b1 = p1[:q], p1[q:]
    t0 = jnp.minimum(a0, b0)
    mm1 = jnp.maximum(a0, b0)
    nn1 = jnp.minimum(a1, b1)
    t1 = jnp.minimum(mm1, nn1)
    t2 = jnp.maximum(mm1, nn1)
    while t0.shape[0] > 8:
        q = t0.shape[0] // 2
        a0, b0 = t0[:q], t0[q:]
        a1, b1 = t1[:q], t1[q:]
        a2, b2 = t2[:q], t2[q:]
        c0 = jnp.minimum(a0, b0)
        mm1 = jnp.maximum(a0, b0)
        nn1 = jnp.minimum(a1, b1)
        c1 = jnp.minimum(mm1, nn1)
        mm2 = jnp.maximum(mm1, nn1)
        nn2 = jnp.minimum(a2, b2)
        c2 = jnp.minimum(mm2, nn2)
        t0, t1, t2 = c0, c1, c2
    allc = jnp.concatenate([t0, t1, t2], axis=0)
    k0 = jnp.min(allc, axis=0, keepdims=True)
    allc1 = jnp.where(allc == k0, big, allc)
    k1 = jnp.min(allc1, axis=0, keepdims=True)
    allc2 = jnp.where(allc1 == k1, big, allc1)
    k2s = jnp.min(allc2, axis=0, keepdims=True)

    def _unkey(k):
        kb_ = jax.lax.bitcast_convert_type(k, jnp.int32) - bias
        return jax.lax.bitcast_convert_type(kb_ & ~lowmask, jnp.float32)

    r0 = 1.0 / (_unkey(k0) + 1e-8)
    r1 = 1.0 / (_unkey(k1) + 1e-8)
    r2 = 1.0 / (_unkey(k2s) + 1e-8)
    rnorm = 1.0 / (r0 + r1 + r2)

    st = jnp.where(key == k0, r0 * rnorm, jnp.float32(0.0))
    st = jnp.where(key == k1, r1 * rnorm, st)
    st = jnp.where(key == k2s, r2 * rnorm, st)

    interp = jnp.dot(kf_ref[0].astype(jnp.bfloat16), st.astype(jnp.bfloat16),
                     preferred_element_type=jnp.float32)
    f = jnp.concatenate([interp.astype(jnp.bfloat16),
                         uf_ref[0].astype(jnp.bfloat16)], axis=0)
    w1b = w1_ref[...].astype(jnp.bfloat16)
    x1 = jnp.dot(w1b, f, preferred_element_type=jnp.float32)
    x1b = x1.astype(jnp.bfloat16)
    x1_ref[0] = x1b

    @pl.when(jnp.logical_and(b == 0, t == 0))
    def _init():
        acc_ref[...] = jnp.zeros_like(acc_ref)

    ones = jnp.ones((tn, 1), dtype=jnp.bfloat16)
    rs = jnp.dot(x1b, ones, preferred_element_type=jnp.float32)
    rss = jnp.dot(x1b * x1b, ones, preferred_element_type=jnp.float32)
    acc_ref[...] += jnp.concatenate([rs, rss], axis=1)


def _bn_coeffs_inkernel(acc, g, bb, ntot):
    mean = acc[:, 0:1] * (1.0 / ntot)
    var = acc[:, 1:2] * (1.0 / ntot) - mean * mean
    sc = g * jax.lax.rsqrt(var + _BN_EPS)
    bi = bb - mean * sc
    return sc, bi


def _pass2_kernel(x1_ref, acc1_ref, g_ref, b_ref, w2_ref,
                  x2_ref, acc_ref, *, ntot):
    b = pl.program_id(0)
    t = pl.program_id(1)
    sc, bi = _bn_coeffs_inkernel(acc1_ref[...], g_ref[...], b_ref[...], ntot)
    y = jnp.maximum(x1_ref[0].astype(jnp.float32) * sc + bi, 0.0)
    w2b = w2_ref[...].astype(jnp.bfloat16)
    x2 = jnp.dot(w2b, y.astype(jnp.bfloat16),
                 preferred_element_type=jnp.float32)
    x2b = x2.astype(jnp.bfloat16)
    x2_ref[0] = x2b

    @pl.when(jnp.logical_and(b == 0, t == 0))
    def _init():
        acc_ref[...] = jnp.zeros_like(acc_ref)

    ones = jnp.ones((x2.shape[1], 1), dtype=jnp.bfloat16)
    rs = jnp.dot(x2b, ones, preferred_element_type=jnp.float32)
    rss = jnp.dot(x2b * x2b, ones, preferred_element_type=jnp.float32)
    acc_ref[...] += jnp.concatenate([rs, rss], axis=1)


def _pass3_kernel(x2_ref, acc2_ref, g_ref, b_ref, out_ref, *, ntot):
    sc, bi = _bn_coeffs_inkernel(acc2_ref[...], g_ref[...], b_ref[...], ntot)
    out_ref[0] = jnp.maximum(x2_ref[0].astype(jnp.float32) * sc + bi, 0.0)


def kernel(unknown, known, unknow_feats, known_feats, W1, g1, b1, W2, g2, b2):
    B, n, _ = unknown.shape
    m = known.shape[1]
    c2 = known_feats.shape[1]
    c1 = unknow_feats.shape[1]
    co1 = W1.shape[0]
    co2 = W2.shape[0]
    tn = _TN
    nt = n // tn
    ntot = float(B * n)

    g1r = g1.reshape(co1, 1)
    b1r = b1.reshape(co1, 1)
    g2r = g2.reshape(co2, 1)
    b2r = b2.reshape(co2, 1)

    grid = (B, nt)
    x1, acc1 = pl.pallas_call(
        functools.partial(_pass1_kernel, m=m),
        grid=grid,
        in_specs=[
            pl.BlockSpec((1, tn, 3), lambda b, t: (b, t, 0)),
            pl.BlockSpec((1, m, 3), lambda b, t: (b, 0, 0)),
            pl.BlockSpec((1, c2, m), lambda b, t: (b, 0, 0)),
            pl.BlockSpec((1, c1, tn), lambda b, t: (b, 0, t)),
            pl.BlockSpec((co1, c1 + c2), lambda b, t: (0, 0)),
        ],
        out_specs=[
            pl.BlockSpec((1, co1, tn), lambda b, t: (b, 0, t)),
            pl.BlockSpec((co1, 2), lambda b, t: (0, 0)),
        ],
        out_shape=[
            jax.ShapeDtypeStruct((B, co1, n), jnp.bfloat16),
            jax.ShapeDtypeStruct((co1, 2), jnp.float32),
        ],
    )(unknown, known, known_feats, unknow_feats, W1)

    x2, acc2 = pl.pallas_call(
        functools.partial(_pass2_kernel, ntot=ntot),
        grid=grid,
        in_specs=[
            pl.BlockSpec((1, co1, tn), lambda b, t: (b, 0, t)),
            pl.BlockSpec((co1, 2), lambda b, t: (0, 0)),
            pl.BlockSpec((co1, 1), lambda b, t: (0, 0)),
            pl.BlockSpec((co1, 1), lambda b, t: (0, 0)),
            pl.BlockSpec((co2, co1), lambda b, t: (0, 0)),
        ],
        out_specs=[
            pl.BlockSpec((1, co2, tn), lambda b, t: (b, 0, t)),
            pl.BlockSpec((co2, 2), lambda b, t: (0, 0)),
        ],
        out_shape=[
            jax.ShapeDtypeStruct((B, co2, n), jnp.bfloat16),
            jax.ShapeDtypeStruct((co2, 2), jnp.float32),
        ],
    )(x1, acc1, g1r, b1r, W2)

    out = pl.pallas_call(
        functools.partial(_pass3_kernel, ntot=ntot),
        grid=grid,
        in_specs=[
            pl.BlockSpec((1, co2, tn), lambda b, t: (b, 0, t)),
            pl.BlockSpec((co2, 2), lambda b, t: (0, 0)),
            pl.BlockSpec((co2, 1), lambda b, t: (0, 0)),
            pl.BlockSpec((co2, 1), lambda b, t: (0, 0)),
        ],
        out_specs=pl.BlockSpec((1, co2, tn), lambda b, t: (b, 0, t)),
        out_shape=jax.ShapeDtypeStruct((B, co2, n), jnp.float32),
    )(x2, acc2, g2r, b2r)

    return out
